# Initial kernel scaffold; baseline (speedup 1.0000x reference)
import jax
import jax.numpy as jnp
from jax.experimental import pallas as pl


def kernel(x, edge_index, edge_attr, W1a, b1a, W1b, b1b, W1c, b1c, W2a, b2a, W2b, b2b, Wha, bha, Whb, bhb):
    raise NotImplementedError("write your pallas kernel here")


# speedup vs baseline: 1.4102x; 1.4102x over previous
import functools

import jax
import jax.numpy as jnp
from jax import lax
from jax.experimental import pallas as pl
from jax.experimental.pallas import tpu as pltpu
from jax.experimental.pallas import tpu_sc as plsc

N_NODES = 10000
N_EDGES = 320000
FRAMES = 8
NT = 4
GES = 16

NC = 2
NS = 16
NW = NC * NS

GRP = 125
N_GRP = N_EDGES // GRP
GPW = N_GRP // NW
GCHUNK = 8
NCHUNK = GPW // GCHUNK

D_G = 32
BE = 8000

_mesh = plsc.VectorSubcoreMesh(core_axis_name="c", subcore_axis_name="s")


@functools.partial(
    pl.kernel,
    mesh=_mesh,
    out_type=jax.ShapeDtypeStruct((N_GRP, GRP, D_G), jnp.float32),
    scratch_types=[
        pltpu.VMEM((GPW, GRP), jnp.int32),
        pltpu.VMEM((GCHUNK, GRP, D_G), jnp.float32),
        pltpu.SemaphoreType.DMA,
    ],
    compiler_params=pltpu.CompilerParams(use_tc_tiling_on_sc=False),
)
def _sc_gather(table_hbm, idx_hbm, out_hbm, idx_v, rows_v, sem):
    wid = lax.axis_index("s") * NC + lax.axis_index("c")
    g0 = wid * GPW
    pltpu.sync_copy(idx_hbm.at[pl.ds(g0, GPW)], idx_v)

    def chunk(c, carry):
        base = c * GCHUNK
        descs = [
            pltpu.async_copy(table_hbm.at[idx_v.at[base + j]], rows_v.at[j], sem)
            for j in range(GCHUNK)
        ]
        for d in descs:
            d.wait()
        pltpu.sync_copy(rows_v, out_hbm.at[pl.ds(g0 + base, GCHUNK)])
        return carry

    lax.fori_loop(0, NCHUNK, chunk, 0)


@functools.partial(
    pl.kernel,
    mesh=_mesh,
    out_type=jax.ShapeDtypeStruct((NC, N_NODES), jnp.float32),
    scratch_types=[
        pltpu.VMEM((GPW, GRP), jnp.int32),
        pltpu.VMEM((GPW, GRP), jnp.float32),
        pltpu.VMEM_SHARED((N_NODES,), jnp.float32),
    ],
    compiler_params=pltpu.CompilerParams(use_tc_tiling_on_sc=False),
)
def _sc_scatter_add(val_hbm, idx_hbm, zeros_hbm, out_hbm, idx_v, val_v, acc_sh):
    cid = lax.axis_index("c")
    sid = lax.axis_index("s")
    wid = sid * NC + cid
    g0 = wid * GPW

    @pl.when(sid == 0)
    def _():
        pltpu.sync_copy(zeros_hbm, acc_sh)

    plsc.subcore_barrier()
    pltpu.sync_copy(idx_hbm.at[pl.ds(g0, GPW)], idx_v)
    pltpu.sync_copy(val_hbm.at[pl.ds(g0, GPW)], val_v)

    def group(j, carry):
        pltpu.sync_copy(val_v.at[j], acc_sh.at[idx_v.at[j]], add=True)
        return carry

    lax.fori_loop(0, GPW, group, 0)
    plsc.subcore_barrier()

    @pl.when(sid == 0)
    def _():
        pltpu.sync_copy(acc_sh, out_hbm.at[cid])


def _init_body(x0_ref, wx_ref, b1a_ref, g_ref):
    g_ref[...] = (
        jnp.dot(x0_ref[...], wx_ref[...], preferred_element_type=jnp.float32)
        + b1a_ref[...]
    )


def _tc_init_g(x0_first, wx, b1a_r):
    return pl.pallas_call(
        _init_body,
        out_shape=jax.ShapeDtypeStruct((N_NODES, D_G), jnp.float32),
    )(x0_first, wx, b1a_r)


def _edge_body(e_ref, ea_ref, w1b_ref, b1b_ref, w1c_ref, b1c_ref, out_ref):
    h = jnp.maximum(e_ref[...], 0.0)
    h = jnp.dot(h, w1b_ref[...], preferred_element_type=jnp.float32) + b1b_ref[...]
    h = jnp.maximum(h, 0.0)
    agg = jnp.tanh(
        jnp.dot(h, w1c_ref[...], preferred_element_type=jnp.float32) + b1c_ref[...]
    )
    nor = jnp.sqrt(jnp.sum(agg * agg, axis=1, keepdims=True))
    agg = agg / jnp.maximum(1.0, nor)
    ea = ea_ref[...]
    r1 = jnp.sum(ea[:, :NT] * agg[:, :NT], axis=-1, keepdims=True)
    r2 = jnp.sum(ea[:, NT:] * agg[:, NT:], axis=-1, keepdims=True)
    r3 = jnp.sum(ea[:, :NT] * agg[:, NT:], axis=-1, keepdims=True)
    r4 = jnp.sum(ea[:, NT:] * agg[:, :NT], axis=-1, keepdims=True)
    out_ref[...] = (r1 + r2) * (r1 + r2) + (r3 - r4) * (r3 - r4)


def _tc_edge(e_flat, ea, w1b, b1b_r, w1c, b1c_r):
    nblk = N_EDGES // BE
    return pl.pallas_call(
        _edge_body,
        grid=(nblk,),
        in_specs=[
            pl.BlockSpec((BE, D_G), lambda i: (i, 0)),
            pl.BlockSpec((BE, 2 * NT), lambda i: (i, 0)),
            pl.BlockSpec((32, 32), lambda i: (0, 0)),
            pl.BlockSpec((1, 32), lambda i: (0, 0)),
            pl.BlockSpec((32, 2 * NT), lambda i: (0, 0)),
            pl.BlockSpec((1, 2 * NT), lambda i: (0, 0)),
        ],
        out_specs=pl.BlockSpec((BE, 1), lambda i: (i, 0)),
        out_shape=jax.ShapeDtypeStruct((N_EDGES, 1), jnp.float32),
    )(e_flat, ea, w1b, b1b_r, w1c, b1c_r)


def _update_body(
    xt_rest_ref, hidden_ref, parts_ref, xn_first_ref,
    w2a_x_ref, w2a_h_ref, w2a_r_ref, b2a_ref, w2b_ref, b2b_ref,
    wha_ref, bha_ref, whb_ref, bhb_ref, wx_ref, wh_ref, b1a_ref,
    hid_out_ref, o_out_ref, g_out_ref,
):
    ones = jnp.ones((NC, 1), jnp.float32)
    aggr = lax.dot_general(
        parts_ref[...], ones, (((0,), (0,)), ((), ())),
        preferred_element_type=jnp.float32,
    )
    hidden = hidden_ref[...]
    t2 = (
        jnp.dot(xt_rest_ref[...], w2a_x_ref[...], preferred_element_type=jnp.float32)
        + jnp.dot(hidden, w2a_h_ref[...], preferred_element_type=jnp.float32)
        + aggr * w2a_r_ref[...]
        + b2a_ref[...]
    )
    h2 = jnp.maximum(t2, 0.0)
    comb = jnp.dot(h2, w2b_ref[...], preferred_element_type=jnp.float32) + b2b_ref[...]
    hidden_new = jnp.tanh(jnp.maximum(comb, 0.0))
    o = jnp.maximum(
        jnp.dot(hidden_new, wha_ref[...], preferred_element_type=jnp.float32)
        + bha_ref[...],
        0.0,
    )
    o = jnp.tanh(jnp.dot(o, whb_ref[...], preferred_element_type=jnp.float32) + bhb_ref[...])
    nor = jnp.sqrt(jnp.sum(o * o, axis=1, keepdims=True))
    o = o / jnp.maximum(1.0, nor)
    hid_out_ref[...] = hidden_new
    o_out_ref[...] = o
    g_out_ref[...] = (
        jnp.dot(xn_first_ref[...], wx_ref[...], preferred_element_type=jnp.float32)
        + jnp.dot(hidden_new, wh_ref[...], preferred_element_type=jnp.float32)
        + b1a_ref[...]
    )


def _tc_update(xt_rest, hidden, parts, xn_first, weights):
    return pl.pallas_call(
        _update_body,
        out_shape=(
            jax.ShapeDtypeStruct((N_NODES, GES), jnp.float32),
            jax.ShapeDtypeStruct((N_NODES, 2 * NT), jnp.float32),
            jax.ShapeDtypeStruct((N_NODES, D_G), jnp.float32),
        ),
    )(xt_rest, hidden, parts, xn_first, *weights)


def kernel(x, edge_index, edge_attr, W1a, b1a, W1b, b1b, W1c, b1c,
           W2a, b2a, W2b, b2b, Wha, bha, Whb, bhb):
    wx = W1a[: 2 * NT]
    wh = W1a[2 * NT:]
    b1a_r = b1a[None, :]
    b1b_r = b1b[None, :]
    b1c_r = b1c[None, :]
    upd_weights = (
        W2a[: 2 * NT],
        W2a[2 * NT: 2 * NT + GES],
        W2a[2 * NT + GES:],
        b2a[None, :],
        W2b, b2b[None, :],
        Wha, bha[None, :],
        Whb, bhb[None, :],
        wx, wh, b1a_r,
    )

    src_g = edge_index[0].reshape(FRAMES, N_GRP, GRP)
    dst_g = edge_index[1].reshape(FRAMES, N_GRP, GRP)
    ea_t = jnp.transpose(edge_attr, (1, 0, 2))
    zeros_nodes = jnp.zeros((N_NODES,), jnp.float32)

    hidden = jnp.zeros((N_NODES, GES), jnp.float32)
    g_tab = _tc_init_g(x[:, 0, : 2 * NT], wx, b1a_r)

    outs = []
    for t in range(FRAMES - 1):
        e_rows = _sc_gather(g_tab, src_g[t])
        rx = _tc_edge(e_rows.reshape(N_EDGES, D_G), ea_t[t],
                      W1b, b1b_r, W1c, b1c_r)
        parts = _sc_scatter_add(rx.reshape(N_GRP, GRP), dst_g[t],
                                zeros_nodes)
        hidden, o, g_tab = _tc_update(
            x[:, t, 2 * NT:], hidden, parts, x[:, t + 1, : 2 * NT], upd_weights)
        outs.append(o)

    return jnp.stack(outs, axis=0)
